# Initial kernel scaffold; baseline (speedup 1.0000x reference)
import jax
import jax.numpy as jnp
from jax.experimental import pallas as pl


def kernel(x, edge_index, W1, b1, W2, b2):
    raise NotImplementedError("write your pallas kernel here")


# speedup vs baseline: 38.7950x; 38.7950x over previous
import functools

import jax
import jax.numpy as jnp
from jax import lax
from jax.experimental import pallas as pl
from jax.experimental.pallas import tpu as pltpu
from jax.experimental.pallas import tpu_sc as plsc

NC = 2
NS = 16
ROWW = 100
G = 8
D = 8
NP = 100096


def _sc_segment_sum(table, src2d, dst2d, zeros_nd):
    n = table.shape[0]
    total_rows = src2d.shape[0]
    rows_per_w = total_rows // (NC * NS)
    n_iter = rows_per_w // G
    assert rows_per_w * NC * NS == total_rows and n_iter * G == rows_per_w
    n_per_tile = n // NS
    assert n_per_tile * NS == n

    mesh = plsc.VectorSubcoreMesh(
        core_axis_name="c", subcore_axis_name="s", num_cores=NC, num_subcores=NS
    )

    @functools.partial(
        pl.kernel,
        out_type=jax.ShapeDtypeStruct((NC, n, D), jnp.float32),
        mesh=mesh,
        scratch_types=[
            pltpu.VMEM((G, ROWW), jnp.int32),
            pltpu.VMEM((G, ROWW), jnp.int32),
            pltpu.VMEM((G, ROWW, D), jnp.float32),
            pltpu.VMEM_SHARED((n, D), jnp.float32),
            pltpu.VMEM_SHARED((n, D), jnp.float32),
            pltpu.SemaphoreType.DMA,
            pltpu.SemaphoreType.DMA,
        ],
        compiler_params=pltpu.CompilerParams(use_tc_tiling_on_sc=False),
    )
    def seg_kernel(table_hbm, src_hbm, dst_hbm, z_hbm, out_hbm,
                   srcbuf, dstbuf, rowbuf, table_sh, acc_sh, isem, gsem):
        c = lax.axis_index("c")
        s = lax.axis_index("s")
        t0 = s * n_per_tile
        pltpu.sync_copy(table_hbm.at[pl.ds(t0, n_per_tile)],
                        table_sh.at[pl.ds(t0, n_per_tile)])
        pltpu.sync_copy(z_hbm.at[pl.ds(t0, n_per_tile)],
                        acc_sh.at[pl.ds(t0, n_per_tile)])
        plsc.subcore_barrier()

        w = c * NS + s
        row0 = w * rows_per_w

        def body(i, carry):
            r0 = row0 + i * G
            cp_s = pltpu.async_copy(src_hbm.at[pl.ds(r0, G)], srcbuf, isem)
            cp_d = pltpu.async_copy(dst_hbm.at[pl.ds(r0, G)], dstbuf, isem)
            cp_s.wait()
            cp_d.wait()
            gathers = [
                pltpu.async_copy(table_sh.at[srcbuf.at[j]], rowbuf.at[j], gsem)
                for j in range(G)
            ]
            for gcp in gathers:
                gcp.wait()
            for j in range(G):
                pltpu.sync_copy(rowbuf.at[j], acc_sh.at[dstbuf.at[j]], add=True)
            return carry

        lax.fori_loop(0, n_iter, body, 0)
        plsc.subcore_barrier()
        pltpu.sync_copy(acc_sh.at[pl.ds(t0, n_per_tile)],
                        out_hbm.at[c, pl.ds(t0, n_per_tile)])

    return seg_kernel(table, src2d, dst2d, zeros_nd)


def _dense_body(acc_ref, w1_ref, b1_ref, w2_ref, out_ref):
    sblk = acc_ref[0] + acc_ref[1]
    h = jnp.dot(sblk, w1_ref[...], preferred_element_type=jnp.float32)
    h = jnp.maximum(h + b1_ref[...], 0.0)
    out_ref[...] = jnp.dot(h, w2_ref[...], preferred_element_type=jnp.float32)


def _dense(acc_pair, w1p, b1r, w2p, rows_blk=2176):
    n = acc_pair.shape[1]
    grid = n // rows_blk
    assert grid * rows_blk == n
    return pl.pallas_call(
        _dense_body,
        grid=(grid,),
        in_specs=[
            pl.BlockSpec((NC, rows_blk, D), lambda i: (0, i, 0)),
            pl.BlockSpec((D, 32), lambda i: (0, 0)),
            pl.BlockSpec((1, 32), lambda i: (0, 0)),
            pl.BlockSpec((32, D), lambda i: (0, 0)),
        ],
        out_specs=pl.BlockSpec((rows_blk, D), lambda i: (i, 0)),
        out_shape=jax.ShapeDtypeStruct((n, D), jnp.float32),
    )(acc_pair, w1p, b1r, w2p)


def _finish_body(acc_ref, b2_ref, out_ref):
    sblk = acc_ref[0] + acc_ref[1]
    out_ref[...] = sblk[:, 0:3] + b2_ref[...]


def _finish(acc_pair, b2r, rows_blk=2176):
    n = acc_pair.shape[1]
    grid = n // rows_blk
    return pl.pallas_call(
        _finish_body,
        grid=(grid,),
        in_specs=[
            pl.BlockSpec((NC, rows_blk, D), lambda i: (0, i, 0)),
            pl.BlockSpec((1, 3), lambda i: (0, 0)),
        ],
        out_specs=pl.BlockSpec((rows_blk, 3), lambda i: (i, 0)),
        out_shape=jax.ShapeDtypeStruct((n, 3), jnp.float32),
    )(acc_pair, b2r)


def kernel(x, edge_index, W1, b1, W2, b2):
    n, f_in = x.shape
    e = edge_index.shape[1]
    f_mid = W1.shape[1]
    assert e % (NC * NS * ROWW * G) == 0

    xp = jnp.zeros((NP, D), jnp.float32).at[:n, :f_in].set(x)
    src2d = edge_index[0].reshape(e // ROWW, ROWW)
    dst2d = edge_index[1].reshape(e // ROWW, ROWW)
    z = jnp.zeros((NP, D), jnp.float32)

    acc1 = _sc_segment_sum(xp, src2d, dst2d, z)

    w1p = jnp.zeros((D, f_mid), jnp.float32).at[:f_in].set(W1)
    w2p = jnp.zeros((f_mid, D), jnp.float32).at[:, :3].set(W2)
    p = _dense(acc1, w1p, b1.reshape(1, f_mid), w2p)

    acc2 = _sc_segment_sum(p, src2d, dst2d, z)
    return _finish(acc2, b2.reshape(1, 3))[:n]


# speedup vs baseline: 57.0008x; 1.4693x over previous
import functools

import jax
import jax.numpy as jnp
from jax import lax
from jax.experimental import pallas as pl
from jax.experimental.pallas import tpu as pltpu
from jax.experimental.pallas import tpu_sc as plsc

NC = 2
NS = 16
ROWW = 80
G = 10
D = 8
NP = 100096


def _sc_segment_sum(table, edge_index, zeros_nd):
    n = table.shape[0]
    e = edge_index.shape[1]
    e_per_w = e // (NC * NS)
    n_iter = e_per_w // (G * ROWW)
    assert e_per_w * NC * NS == e and n_iter * G * ROWW == e_per_w
    n_per_tile = n // NS
    assert n_per_tile * NS == n

    mesh = plsc.VectorSubcoreMesh(
        core_axis_name="c", subcore_axis_name="s", num_cores=NC, num_subcores=NS
    )

    @functools.partial(
        pl.kernel,
        out_type=jax.ShapeDtypeStruct((NC, n, D), jnp.float32),
        mesh=mesh,
        scratch_types=[
            pltpu.VMEM((G, ROWW), jnp.int32),
            pltpu.VMEM((G, ROWW), jnp.int32),
            pltpu.VMEM((G, ROWW, D), jnp.float32),
            pltpu.VMEM_SHARED((n, D), jnp.float32),
            pltpu.VMEM_SHARED((n, D), jnp.float32),
            pltpu.SemaphoreType.DMA,
            pltpu.SemaphoreType.DMA,
        ],
        compiler_params=pltpu.CompilerParams(use_tc_tiling_on_sc=False),
    )
    def seg_kernel(table_hbm, edge_hbm, z_hbm, out_hbm,
                   srcbuf, dstbuf, rowbuf, table_sh, acc_sh, isem, gsem):
        c = lax.axis_index("c")
        s = lax.axis_index("s")
        t0 = s * n_per_tile
        pltpu.sync_copy(table_hbm.at[pl.ds(t0, n_per_tile)],
                        table_sh.at[pl.ds(t0, n_per_tile)])
        pltpu.sync_copy(z_hbm.at[pl.ds(t0, n_per_tile)],
                        acc_sh.at[pl.ds(t0, n_per_tile)])
        plsc.subcore_barrier()

        w = c * NS + s
        base = w * e_per_w

        def body(i, carry):
            off = base + i * (G * ROWW)
            loads = []
            for j in range(G):
                loads.append(pltpu.async_copy(
                    edge_hbm.at[0, pl.ds(off + j * ROWW, ROWW)],
                    srcbuf.at[j], isem))
                loads.append(pltpu.async_copy(
                    edge_hbm.at[1, pl.ds(off + j * ROWW, ROWW)],
                    dstbuf.at[j], isem))
            for cp in loads:
                cp.wait()
            gathers = [
                pltpu.async_copy(table_sh.at[srcbuf.at[j]], rowbuf.at[j], gsem)
                for j in range(G)
            ]
            for gcp in gathers:
                gcp.wait()
            for j in range(G):
                pltpu.sync_copy(rowbuf.at[j], acc_sh.at[dstbuf.at[j]], add=True)
            return carry

        lax.fori_loop(0, n_iter, body, 0)
        plsc.subcore_barrier()
        pltpu.sync_copy(acc_sh.at[pl.ds(t0, n_per_tile)],
                        out_hbm.at[c, pl.ds(t0, n_per_tile)])

    return seg_kernel(table, edge_index, zeros_nd)


def _dense_body(acc_ref, w1_ref, b1_ref, w2_ref, out_ref):
    sblk = acc_ref[0] + acc_ref[1]
    h = jnp.dot(sblk, w1_ref[...], preferred_element_type=jnp.float32)
    h = jnp.maximum(h + b1_ref[...], 0.0)
    out_ref[...] = jnp.dot(h, w2_ref[...], preferred_element_type=jnp.float32)


def _dense(acc_pair, w1p, b1r, w2p, rows_blk=2176):
    n = acc_pair.shape[1]
    grid = n // rows_blk
    assert grid * rows_blk == n
    return pl.pallas_call(
        _dense_body,
        grid=(grid,),
        in_specs=[
            pl.BlockSpec((NC, rows_blk, D), lambda i: (0, i, 0)),
            pl.BlockSpec((D, 32), lambda i: (0, 0)),
            pl.BlockSpec((1, 32), lambda i: (0, 0)),
            pl.BlockSpec((32, D), lambda i: (0, 0)),
        ],
        out_specs=pl.BlockSpec((rows_blk, D), lambda i: (i, 0)),
        out_shape=jax.ShapeDtypeStruct((n, D), jnp.float32),
    )(acc_pair, w1p, b1r, w2p)


def _finish_body(acc_ref, b2_ref, out_ref):
    sblk = acc_ref[0] + acc_ref[1]
    out_ref[...] = sblk[:, 0:3] + b2_ref[...]


def _finish(acc_pair, b2r, rows_blk=2176):
    n = acc_pair.shape[1]
    grid = n // rows_blk
    return pl.pallas_call(
        _finish_body,
        grid=(grid,),
        in_specs=[
            pl.BlockSpec((NC, rows_blk, D), lambda i: (0, i, 0)),
            pl.BlockSpec((1, 3), lambda i: (0, 0)),
        ],
        out_specs=pl.BlockSpec((rows_blk, 3), lambda i: (i, 0)),
        out_shape=jax.ShapeDtypeStruct((n, 3), jnp.float32),
    )(acc_pair, b2r)


def kernel(x, edge_index, W1, b1, W2, b2):
    n, f_in = x.shape
    e = edge_index.shape[1]
    f_mid = W1.shape[1]
    assert e % (NC * NS * ROWW * G) == 0

    xp = jnp.zeros((NP, D), jnp.float32).at[:n, :f_in].set(x)
    z = jnp.zeros((NP, D), jnp.float32)

    acc1 = _sc_segment_sum(xp, edge_index, z)

    w1p = jnp.zeros((D, f_mid), jnp.float32).at[:f_in].set(W1)
    w2p = jnp.zeros((f_mid, D), jnp.float32).at[:, :3].set(W2)
    p = _dense(acc1, w1p, b1.reshape(1, f_mid), w2p)

    acc2 = _sc_segment_sum(p, edge_index, z)
    return _finish(acc2, b2.reshape(1, 3))[:n]


# speedup vs baseline: 96.9486x; 1.7008x over previous
import functools

import jax
import jax.numpy as jnp
from jax import lax
from jax.experimental import pallas as pl
from jax.experimental.pallas import tpu as pltpu
from jax.experimental.pallas import tpu_sc as plsc

NC = 2
NS = 16
ROWW = 80
G = 10
D = 8
NP = 100096


def _sc_segment_sum(table, edge_index, zeros_nd):
    n = table.shape[0]
    e = edge_index.shape[1]
    e_per_w = e // (NC * NS)
    n_iter = e_per_w // (G * ROWW)
    assert e_per_w * NC * NS == e and n_iter * G * ROWW == e_per_w
    n_per_tile = n // NS
    assert n_per_tile * NS == n

    mesh = plsc.VectorSubcoreMesh(
        core_axis_name="c", subcore_axis_name="s", num_cores=NC, num_subcores=NS
    )

    assert n_iter % 2 == 0 and n_iter >= 4
    chunk = G * ROWW

    @functools.partial(
        pl.kernel,
        out_type=jax.ShapeDtypeStruct((NC, n, D), jnp.float32),
        mesh=mesh,
        scratch_types=[
            pltpu.VMEM((2, G, ROWW), jnp.int32),
            pltpu.VMEM((2, G, ROWW), jnp.int32),
            pltpu.VMEM((2, G, ROWW, D), jnp.float32),
            pltpu.VMEM_SHARED((n, D), jnp.float32),
            pltpu.VMEM_SHARED((n, D), jnp.float32),
            pltpu.SemaphoreType.DMA,
            pltpu.SemaphoreType.DMA,
            pltpu.SemaphoreType.DMA,
            pltpu.SemaphoreType.DMA,
            pltpu.SemaphoreType.DMA,
            pltpu.SemaphoreType.DMA,
            pltpu.SemaphoreType.DMA,
            pltpu.SemaphoreType.DMA,
        ],
        compiler_params=pltpu.CompilerParams(use_tc_tiling_on_sc=False),
    )
    def seg_kernel(table_hbm, edge_hbm, z_hbm, out_hbm,
                   srcbuf, dstbuf, rowbuf, table_sh, acc_sh,
                   isem0, isem1, dsem0, dsem1, gsem0, gsem1, ssem0, ssem1):
        isem = (isem0, isem1)
        dsem = (dsem0, dsem1)
        gsem = (gsem0, gsem1)
        ssem = (ssem0, ssem1)
        c = lax.axis_index("c")
        s = lax.axis_index("s")
        t0 = s * n_per_tile
        pltpu.sync_copy(table_hbm.at[pl.ds(t0, n_per_tile)],
                        table_sh.at[pl.ds(t0, n_per_tile)])
        pltpu.sync_copy(z_hbm.at[pl.ds(t0, n_per_tile)],
                        acc_sh.at[pl.ds(t0, n_per_tile)])
        plsc.subcore_barrier()

        w = c * NS + s
        base = w * e_per_w

        def fire_src(k, b):
            off = base + k * chunk
            for j in range(G):
                pltpu.async_copy(edge_hbm.at[0, pl.ds(off + j * ROWW, ROWW)],
                                 srcbuf.at[b, j], isem[b])

        def wait_src(k, b):
            off = base + k * chunk
            for j in range(G):
                pltpu.make_async_copy(
                    edge_hbm.at[0, pl.ds(off + j * ROWW, ROWW)],
                    srcbuf.at[b, j], isem[b]).wait()

        def drain_scatters(b):
            for j in range(G):
                pltpu.make_async_copy(z_hbm.at[pl.ds(0, ROWW)],
                                      rowbuf.at[b, j], ssem[b]).wait()

        def body(k, b, first):
            off = base + k * chunk
            if not first:
                drain_scatters(b)
            dloads = [pltpu.async_copy(
                edge_hbm.at[1, pl.ds(off + j * ROWW, ROWW)],
                dstbuf.at[b, j], dsem[b]) for j in range(G)]
            wait_src(k, b)
            gathers = [pltpu.async_copy(table_sh.at[srcbuf.at[b, j]],
                                        rowbuf.at[b, j], gsem[b])
                       for j in range(G)]
            fire_src(k + 1, 1 - b)
            for gcp in gathers:
                gcp.wait()
            for cp in dloads:
                cp.wait()
            for j in range(G):
                pltpu.async_copy(rowbuf.at[b, j], acc_sh.at[dstbuf.at[b, j]],
                                 ssem[b], add=True)

        fire_src(0, 0)
        body(0, 0, True)
        body(1, 1, True)

        def loop_body(i, carry):
            k = 2 * i
            body(k, 0, False)
            body(k + 1, 1, False)
            return carry

        lax.fori_loop(1, n_iter // 2, loop_body, 0)
        drain_scatters(0)
        drain_scatters(1)
        wait_src(n_iter, 0)
        plsc.subcore_barrier()
        pltpu.sync_copy(acc_sh.at[pl.ds(t0, n_per_tile)],
                        out_hbm.at[c, pl.ds(t0, n_per_tile)])

    return seg_kernel(table, edge_index, zeros_nd)


def _dense_body(acc_ref, w1_ref, b1_ref, w2_ref, out_ref):
    sblk = acc_ref[0] + acc_ref[1]
    h = jnp.dot(sblk, w1_ref[...], preferred_element_type=jnp.float32)
    h = jnp.maximum(h + b1_ref[...], 0.0)
    out_ref[...] = jnp.dot(h, w2_ref[...], preferred_element_type=jnp.float32)


def _dense(acc_pair, w1p, b1r, w2p, rows_blk=2176):
    n = acc_pair.shape[1]
    grid = n // rows_blk
    assert grid * rows_blk == n
    return pl.pallas_call(
        _dense_body,
        grid=(grid,),
        in_specs=[
            pl.BlockSpec((NC, rows_blk, D), lambda i: (0, i, 0)),
            pl.BlockSpec((D, 32), lambda i: (0, 0)),
            pl.BlockSpec((1, 32), lambda i: (0, 0)),
            pl.BlockSpec((32, D), lambda i: (0, 0)),
        ],
        out_specs=pl.BlockSpec((rows_blk, D), lambda i: (i, 0)),
        out_shape=jax.ShapeDtypeStruct((n, D), jnp.float32),
    )(acc_pair, w1p, b1r, w2p)


def _finish_body(acc_ref, b2_ref, out_ref):
    sblk = acc_ref[0] + acc_ref[1]
    out_ref[...] = sblk[:, 0:3] + b2_ref[...]


def _finish(acc_pair, b2r, rows_blk=2176):
    n = acc_pair.shape[1]
    grid = n // rows_blk
    return pl.pallas_call(
        _finish_body,
        grid=(grid,),
        in_specs=[
            pl.BlockSpec((NC, rows_blk, D), lambda i: (0, i, 0)),
            pl.BlockSpec((1, 3), lambda i: (0, 0)),
        ],
        out_specs=pl.BlockSpec((rows_blk, 3), lambda i: (i, 0)),
        out_shape=jax.ShapeDtypeStruct((n, 3), jnp.float32),
    )(acc_pair, b2r)


def kernel(x, edge_index, W1, b1, W2, b2):
    n, f_in = x.shape
    e = edge_index.shape[1]
    f_mid = W1.shape[1]
    assert e % (NC * NS * ROWW * G) == 0

    xp = jnp.zeros((NP, D), jnp.float32).at[:n, :f_in].set(x)
    z = jnp.zeros((NP, D), jnp.float32)

    acc1 = _sc_segment_sum(xp, edge_index, z)

    w1p = jnp.zeros((D, f_mid), jnp.float32).at[:f_in].set(W1)
    w2p = jnp.zeros((f_mid, D), jnp.float32).at[:, :3].set(W2)
    p = _dense(acc1, w1p, b1.reshape(1, f_mid), w2p)

    acc2 = _sc_segment_sum(p, edge_index, z)
    return _finish(acc2, b2.reshape(1, 3))[:n]


# speedup vs baseline: 108.8711x; 1.1230x over previous
import functools

import jax
import jax.numpy as jnp
from jax import lax
from jax.experimental import pallas as pl
from jax.experimental.pallas import tpu as pltpu
from jax.experimental.pallas import tpu_sc as plsc

NC = 2
NS = 16
ROWW = 80
G = 10
D = 8
NP = 100352


def _sc_segment_sum(table, edge_index, zeros_nd):
    n = table.shape[0]
    e = edge_index.shape[1]
    e_per_w = e // (NC * NS)
    n_iter = e_per_w // (G * ROWW)
    assert e_per_w * NC * NS == e and n_iter * G * ROWW == e_per_w
    n_per_tile = n // NS
    assert n_per_tile * NS == n

    mesh = plsc.VectorSubcoreMesh(
        core_axis_name="c", subcore_axis_name="s", num_cores=NC, num_subcores=NS
    )

    assert n_iter % 2 == 0 and n_iter >= 4
    chunk = G * ROWW

    @functools.partial(
        pl.kernel,
        out_type=jax.ShapeDtypeStruct((NC, n, D), jnp.float32),
        mesh=mesh,
        scratch_types=[
            pltpu.VMEM((2, G, ROWW), jnp.int32),
            pltpu.VMEM((2, G, ROWW), jnp.int32),
            pltpu.VMEM((2, G, ROWW, D), jnp.float32),
            pltpu.VMEM_SHARED((n, D), jnp.float32),
            pltpu.VMEM_SHARED((n, D), jnp.float32),
            pltpu.SemaphoreType.DMA,
            pltpu.SemaphoreType.DMA,
            pltpu.SemaphoreType.DMA,
            pltpu.SemaphoreType.DMA,
            pltpu.SemaphoreType.DMA,
            pltpu.SemaphoreType.DMA,
            pltpu.SemaphoreType.DMA,
            pltpu.SemaphoreType.DMA,
        ],
        compiler_params=pltpu.CompilerParams(use_tc_tiling_on_sc=False),
    )
    def seg_kernel(table_hbm, edge_hbm, z_hbm, out_hbm,
                   srcbuf, dstbuf, rowbuf, table_sh, acc_sh,
                   isem0, isem1, dsem0, dsem1, gsem0, gsem1, ssem0, ssem1):
        isem = (isem0, isem1)
        dsem = (dsem0, dsem1)
        gsem = (gsem0, gsem1)
        ssem = (ssem0, ssem1)
        c = lax.axis_index("c")
        s = lax.axis_index("s")
        t0 = s * n_per_tile
        pltpu.sync_copy(table_hbm.at[pl.ds(t0, n_per_tile)],
                        table_sh.at[pl.ds(t0, n_per_tile)])
        pltpu.sync_copy(z_hbm.at[pl.ds(t0, n_per_tile)],
                        acc_sh.at[pl.ds(t0, n_per_tile)])
        plsc.subcore_barrier()

        w = c * NS + s
        base = w * e_per_w

        def fire_src(k, b):
            off = base + k * chunk
            for j in range(G):
                pltpu.async_copy(edge_hbm.at[0, pl.ds(off + j * ROWW, ROWW)],
                                 srcbuf.at[b, j], isem[b])

        def wait_src(k, b):
            off = base + k * chunk
            for j in range(G):
                pltpu.make_async_copy(
                    edge_hbm.at[0, pl.ds(off + j * ROWW, ROWW)],
                    srcbuf.at[b, j], isem[b]).wait()

        def drain_scatters(b):
            for j in range(G):
                pltpu.make_async_copy(z_hbm.at[pl.ds(0, ROWW)],
                                      rowbuf.at[b, j], ssem[b]).wait()

        def body(k, b, first):
            off = base + k * chunk
            if not first:
                drain_scatters(b)
            dloads = [pltpu.async_copy(
                edge_hbm.at[1, pl.ds(off + j * ROWW, ROWW)],
                dstbuf.at[b, j], dsem[b]) for j in range(G)]
            wait_src(k, b)
            gathers = [pltpu.async_copy(table_sh.at[srcbuf.at[b, j]],
                                        rowbuf.at[b, j], gsem[b])
                       for j in range(G)]
            fire_src(k + 1, 1 - b)
            for gcp in gathers:
                gcp.wait()
            for cp in dloads:
                cp.wait()
            for j in range(G):
                pltpu.async_copy(rowbuf.at[b, j], acc_sh.at[dstbuf.at[b, j]],
                                 ssem[b], add=True)

        fire_src(0, 0)
        body(0, 0, True)
        body(1, 1, True)

        def loop_body(i, carry):
            k = 2 * i
            body(k, 0, False)
            body(k + 1, 1, False)
            return carry

        lax.fori_loop(1, n_iter // 2, loop_body, 0)
        drain_scatters(0)
        drain_scatters(1)
        wait_src(n_iter, 0)
        plsc.subcore_barrier()
        pltpu.sync_copy(acc_sh.at[pl.ds(t0, n_per_tile)],
                        out_hbm.at[c, pl.ds(t0, n_per_tile)])

    return seg_kernel(table, edge_index, zeros_nd)


def _sc_dense(acc_pair, wsplat, f_in, f_mid):
    n = acc_pair.shape[1]
    per_w = n // (NC * NS)
    groups = per_w // 16
    assert per_w * NC * NS == n and groups * 16 == per_w
    nw = wsplat.shape[0]
    b1_off = D * f_mid
    w2_off = b1_off + f_mid

    mesh = plsc.VectorSubcoreMesh(
        core_axis_name="c", subcore_axis_name="s", num_cores=NC, num_subcores=NS
    )

    @functools.partial(
        pl.kernel,
        out_type=jax.ShapeDtypeStruct((n, D), jnp.float32),
        mesh=mesh,
        scratch_types=[
            pltpu.VMEM((per_w, D), jnp.float32),
            pltpu.VMEM((per_w, D), jnp.float32),
            pltpu.VMEM((per_w, D), jnp.float32),
            pltpu.VMEM((nw, 16), jnp.float32),
        ],
        compiler_params=pltpu.CompilerParams(use_tc_tiling_on_sc=False,
                                             needs_layout_passes=False),
    )
    def dense_kernel(acc_hbm, ws_hbm, p_hbm, a0buf, a1buf, pbuf, wsv):
        c = lax.axis_index("c")
        s = lax.axis_index("s")
        w = c * NS + s
        row0 = w * per_w
        pltpu.sync_copy(acc_hbm.at[0, pl.ds(row0, per_w)], a0buf)
        pltpu.sync_copy(acc_hbm.at[1, pl.ds(row0, per_w)], a1buf)
        pltpu.sync_copy(ws_hbm, wsv)

        iota = lax.iota(jnp.int32, 16)

        def body(g, carry):
            rows = g * 16 + iota
            cols = []
            for k in range(f_in):
                ck = jnp.full((16,), k, jnp.int32)
                cols.append(plsc.load_gather(a0buf, [rows, ck])
                            + plsc.load_gather(a1buf, [rows, ck]))
            p0 = jnp.zeros((16,), jnp.float32)
            p1 = jnp.zeros((16,), jnp.float32)
            p2 = jnp.zeros((16,), jnp.float32)
            for j in range(f_mid):
                h = wsv[b1_off + j]
                for k in range(f_in):
                    h = h + cols[k] * wsv[k * f_mid + j]
                h = jnp.maximum(h, 0.0)
                p0 = p0 + h * wsv[w2_off + j * 4 + 0]
                p1 = p1 + h * wsv[w2_off + j * 4 + 1]
                p2 = p2 + h * wsv[w2_off + j * 4 + 2]
            plsc.store_scatter(pbuf, [rows, jnp.full((16,), 0, jnp.int32)], p0)
            plsc.store_scatter(pbuf, [rows, jnp.full((16,), 1, jnp.int32)], p1)
            plsc.store_scatter(pbuf, [rows, jnp.full((16,), 2, jnp.int32)], p2)
            return carry

        lax.fori_loop(0, groups, body, 0)
        pltpu.sync_copy(pbuf, p_hbm.at[pl.ds(row0, per_w)])

    return dense_kernel(acc_pair, wsplat)


def _sc_finish(acc_pair, bpat):
    n = acc_pair.shape[1]
    per_w = n // (NC * NS)
    nv = per_w * D // 16
    out_w = per_w * 3

    mesh = plsc.VectorSubcoreMesh(
        core_axis_name="c", subcore_axis_name="s", num_cores=NC, num_subcores=NS
    )

    @functools.partial(
        pl.kernel,
        out_type=jax.ShapeDtypeStruct((n * 3,), jnp.float32),
        mesh=mesh,
        scratch_types=[
            pltpu.VMEM((per_w, D), jnp.float32),
            pltpu.VMEM((per_w, D), jnp.float32),
            pltpu.VMEM((out_w + 16,), jnp.float32),
            pltpu.VMEM((16,), jnp.float32),
        ],
        compiler_params=pltpu.CompilerParams(use_tc_tiling_on_sc=False,
                                             needs_layout_passes=False),
    )
    def fin_kernel(acc_hbm, bpat_hbm, out_hbm, a0buf, a1buf, obuf, bbuf):
        c = lax.axis_index("c")
        s = lax.axis_index("s")
        w = c * NS + s
        row0 = w * per_w
        pltpu.sync_copy(acc_hbm.at[0, pl.ds(row0, per_w)], a0buf)
        pltpu.sync_copy(acc_hbm.at[1, pl.ds(row0, per_w)], a1buf)
        pltpu.sync_copy(bpat_hbm, bbuf)

        iota = lax.iota(jnp.int32, 16)
        colb = iota & 7
        rsel = jnp.where(iota >= 8, 1, 0).astype(jnp.int32)
        mask = colb < 3
        bvec = bbuf[...]

        def body(g, carry):
            rows = 2 * g + rsel
            v = (plsc.load_gather(a0buf, [rows, colb])
                 + plsc.load_gather(a1buf, [rows, colb]) + bvec)
            plsc.store_compressed(obuf.at[pl.ds(g * 6, 16)], v, mask=mask)
            return carry

        lax.fori_loop(0, nv, body, 0)
        pltpu.sync_copy(obuf.at[pl.ds(0, out_w)],
                        out_hbm.at[pl.ds(w * out_w, out_w)])

    return fin_kernel(acc_pair, bpat)


def kernel(x, edge_index, W1, b1, W2, b2):
    n, f_in = x.shape
    e = edge_index.shape[1]
    f_mid = W1.shape[1]
    assert e % (NC * NS * ROWW * G) == 0

    xp = jnp.zeros((NP, D), jnp.float32).at[:n, :f_in].set(x)
    z = jnp.zeros((NP, D), jnp.float32)

    acc1 = _sc_segment_sum(xp, edge_index, z)

    w1p = jnp.zeros((D, f_mid), jnp.float32).at[:f_in].set(W1)
    w2c = jnp.zeros((f_mid, 4), jnp.float32).at[:, :3].set(W2)
    ws = jnp.concatenate([w1p.reshape(-1), b1, w2c.reshape(-1)])
    wsplat = jnp.tile(ws[:, None], (1, 16))
    p = _sc_dense(acc1, wsplat, f_in, f_mid)

    acc2 = _sc_segment_sum(p, edge_index, z)

    bpat = jnp.tile(jnp.concatenate([b2, jnp.zeros((5,), jnp.float32)]), 2)
    out_flat = _sc_finish(acc2, bpat)
    return out_flat.reshape(NP, 3)[:n]


# speedup vs baseline: 123.2625x; 1.1322x over previous
import functools

import jax
import jax.numpy as jnp
from jax import lax
from jax.experimental import pallas as pl
from jax.experimental.pallas import tpu as pltpu
from jax.experimental.pallas import tpu_sc as plsc

NC = 2
NS = 16
ROWW = 80
G = 10
D = 8
NP = 100352


def _sc_segment_sum(table, edge_index, zeros_nd):
    n = table.shape[0]
    e = edge_index.shape[1]
    e_per_w = e // (NC * NS)
    n_iter = e_per_w // (G * ROWW)
    assert e_per_w * NC * NS == e and n_iter * G * ROWW == e_per_w
    n_per_tile = n // NS
    assert n_per_tile * NS == n

    mesh = plsc.VectorSubcoreMesh(
        core_axis_name="c", subcore_axis_name="s", num_cores=NC, num_subcores=NS
    )

    assert n_iter % 2 == 0 and n_iter >= 4
    chunk = G * ROWW

    @functools.partial(
        pl.kernel,
        out_type=jax.ShapeDtypeStruct((NC, n, D), jnp.float32),
        mesh=mesh,
        scratch_types=[
            pltpu.VMEM((2, G, ROWW), jnp.int32),
            pltpu.VMEM((2, G, ROWW), jnp.int32),
            pltpu.VMEM((2, G, ROWW, D), jnp.float32),
            pltpu.VMEM_SHARED((n, D), jnp.float32),
            pltpu.VMEM_SHARED((n, D), jnp.float32),
            pltpu.SemaphoreType.DMA,
            pltpu.SemaphoreType.DMA,
            pltpu.SemaphoreType.DMA,
            pltpu.SemaphoreType.DMA,
            pltpu.SemaphoreType.DMA,
            pltpu.SemaphoreType.DMA,
            pltpu.SemaphoreType.DMA,
            pltpu.SemaphoreType.DMA,
        ],
        compiler_params=pltpu.CompilerParams(use_tc_tiling_on_sc=False),
    )
    def seg_kernel(table_hbm, edge_hbm, z_hbm, out_hbm,
                   srcbuf, dstbuf, rowbuf, table_sh, acc_sh,
                   isem0, isem1, dsem0, dsem1, gsem0, gsem1, ssem0, ssem1):
        isem = (isem0, isem1)
        dsem = (dsem0, dsem1)
        gsem = (gsem0, gsem1)
        ssem = (ssem0, ssem1)
        c = lax.axis_index("c")
        s = lax.axis_index("s")
        t0 = s * n_per_tile
        pltpu.sync_copy(table_hbm.at[pl.ds(t0, n_per_tile)],
                        table_sh.at[pl.ds(t0, n_per_tile)])
        pltpu.sync_copy(z_hbm.at[pl.ds(t0, n_per_tile)],
                        acc_sh.at[pl.ds(t0, n_per_tile)])
        plsc.subcore_barrier()

        w = c * NS + s
        base = w * e_per_w

        def fire_src(k, b):
            off = base + k * chunk
            for j in range(G):
                pltpu.async_copy(edge_hbm.at[0, pl.ds(off + j * ROWW, ROWW)],
                                 srcbuf.at[b, j], isem[b])

        def wait_src(k, b):
            off = base + k * chunk
            for j in range(G):
                pltpu.make_async_copy(
                    edge_hbm.at[0, pl.ds(off + j * ROWW, ROWW)],
                    srcbuf.at[b, j], isem[b]).wait()

        def drain_scatters(b):
            for j in range(G):
                pltpu.make_async_copy(z_hbm.at[pl.ds(0, ROWW)],
                                      rowbuf.at[b, j], ssem[b]).wait()

        def body(k, b, first):
            off = base + k * chunk
            if not first:
                drain_scatters(b)
            dloads = [pltpu.async_copy(
                edge_hbm.at[1, pl.ds(off + j * ROWW, ROWW)],
                dstbuf.at[b, j], dsem[b]) for j in range(G)]
            wait_src(k, b)
            gathers = [pltpu.async_copy(table_sh.at[srcbuf.at[b, j]],
                                        rowbuf.at[b, j], gsem[b])
                       for j in range(G)]
            fire_src(k + 1, 1 - b)
            for gcp in gathers:
                gcp.wait()
            for cp in dloads:
                cp.wait()
            for j in range(G):
                pltpu.async_copy(rowbuf.at[b, j], acc_sh.at[dstbuf.at[b, j]],
                                 ssem[b], add=True)

        fire_src(0, 0)
        body(0, 0, True)
        body(1, 1, True)

        def loop_body(i, carry):
            k = 2 * i
            body(k, 0, False)
            body(k + 1, 1, False)
            return carry

        lax.fori_loop(1, n_iter // 2, loop_body, 0)
        drain_scatters(0)
        drain_scatters(1)
        wait_src(n_iter, 0)
        plsc.subcore_barrier()
        pltpu.sync_copy(acc_sh.at[pl.ds(t0, n_per_tile)],
                        out_hbm.at[c, pl.ds(t0, n_per_tile)])

    return seg_kernel(table, edge_index, zeros_nd)


def _sc_dense(acc_pair, wsplat, f_in, f_mid):
    n = acc_pair.shape[1]
    per_w = n // (NC * NS)
    groups = per_w // 16
    assert per_w * NC * NS == n and groups * 16 == per_w
    nw = wsplat.shape[0]
    b1_off = D * f_mid
    w2_off = b1_off + f_mid

    mesh = plsc.VectorSubcoreMesh(
        core_axis_name="c", subcore_axis_name="s", num_cores=NC, num_subcores=NS
    )

    @functools.partial(
        pl.kernel,
        out_type=jax.ShapeDtypeStruct((n, D), jnp.float32),
        mesh=mesh,
        scratch_types=[
            pltpu.VMEM((per_w, D), jnp.float32),
            pltpu.VMEM((per_w, D), jnp.float32),
            pltpu.VMEM((per_w, D), jnp.float32),
            pltpu.VMEM((nw, 16), jnp.float32),
        ],
        compiler_params=pltpu.CompilerParams(use_tc_tiling_on_sc=False,
                                             needs_layout_passes=False),
    )
    def dense_kernel(acc_hbm, ws_hbm, p_hbm, a0buf, a1buf, pbuf, wsv):
        c = lax.axis_index("c")
        s = lax.axis_index("s")
        w = c * NS + s
        row0 = w * per_w
        pltpu.sync_copy(acc_hbm.at[0, pl.ds(row0, per_w)], a0buf)
        pltpu.sync_copy(acc_hbm.at[1, pl.ds(row0, per_w)], a1buf)
        pltpu.sync_copy(ws_hbm, wsv)

        iota = lax.iota(jnp.int32, 16)

        def tsum(terms):
            while len(terms) > 1:
                nxt = [terms[i] + terms[i + 1] for i in range(0, len(terms) - 1, 2)]
                if len(terms) % 2:
                    nxt.append(terms[-1])
                terms = nxt
            return terms[0]

        def body(g2, carry):
            rows_pair = [2 * g2 * 16 + iota, 2 * g2 * 16 + 16 + iota]
            cols_pair = []
            for rows in rows_pair:
                cols = []
                for k in range(f_in):
                    ck = jnp.full((16,), k, jnp.int32)
                    cols.append(plsc.load_gather(a0buf, [rows, ck])
                                + plsc.load_gather(a1buf, [rows, ck]))
                cols_pair.append(cols)
            acc = [[jnp.zeros((16,), jnp.float32) for _ in range(4)]
                   for _ in range(3)]
            for j in range(f_mid):
                wb = wsv[b1_off + j]
                wk = [wsv[k * f_mid + j] for k in range(f_in)]
                w2 = [wsv[w2_off + j * 4 + m] for m in range(3)]
                for gi, cols in enumerate(cols_pair):
                    h = jnp.maximum(tsum([wb] + [cols[k] * wk[k]
                                                 for k in range(f_in)]), 0.0)
                    sl = 2 * gi + (j & 1)
                    for m in range(3):
                        acc[m][sl] = acc[m][sl] + h * w2[m]
            for gi, rows in enumerate(rows_pair):
                for m in range(3):
                    plsc.store_scatter(
                        pbuf, [rows, jnp.full((16,), m, jnp.int32)],
                        acc[m][2 * gi] + acc[m][2 * gi + 1])
            return carry

        lax.fori_loop(0, groups // 2, body, 0)
        pltpu.sync_copy(pbuf, p_hbm.at[pl.ds(row0, per_w)])

    return dense_kernel(acc_pair, wsplat)


def _sc_finish(acc_pair, bpat):
    n = acc_pair.shape[1]
    per_w = n // (NC * NS)
    nv = per_w * D // 16
    out_w = per_w * 3

    mesh = plsc.VectorSubcoreMesh(
        core_axis_name="c", subcore_axis_name="s", num_cores=NC, num_subcores=NS
    )

    @functools.partial(
        pl.kernel,
        out_type=jax.ShapeDtypeStruct((n * 3,), jnp.float32),
        mesh=mesh,
        scratch_types=[
            pltpu.VMEM((per_w, D), jnp.float32),
            pltpu.VMEM((per_w, D), jnp.float32),
            pltpu.VMEM((out_w + 16,), jnp.float32),
            pltpu.VMEM((16,), jnp.float32),
        ],
        compiler_params=pltpu.CompilerParams(use_tc_tiling_on_sc=False,
                                             needs_layout_passes=False),
    )
    def fin_kernel(acc_hbm, bpat_hbm, out_hbm, a0buf, a1buf, obuf, bbuf):
        c = lax.axis_index("c")
        s = lax.axis_index("s")
        w = c * NS + s
        row0 = w * per_w
        pltpu.sync_copy(acc_hbm.at[0, pl.ds(row0, per_w)], a0buf)
        pltpu.sync_copy(acc_hbm.at[1, pl.ds(row0, per_w)], a1buf)
        pltpu.sync_copy(bpat_hbm, bbuf)

        iota = lax.iota(jnp.int32, 16)
        colb = iota & 7
        rsel = jnp.where(iota >= 8, 1, 0).astype(jnp.int32)
        mask = colb < 3
        bvec = bbuf[...]

        def body(g, carry):
            rows = 2 * g + rsel
            v = (plsc.load_gather(a0buf, [rows, colb])
                 + plsc.load_gather(a1buf, [rows, colb]) + bvec)
            plsc.store_compressed(obuf.at[pl.ds(g * 6, 16)], v, mask=mask)
            return carry

        lax.fori_loop(0, nv, body, 0)
        pltpu.sync_copy(obuf.at[pl.ds(0, out_w)],
                        out_hbm.at[pl.ds(w * out_w, out_w)])

    return fin_kernel(acc_pair, bpat)


def kernel(x, edge_index, W1, b1, W2, b2):
    n, f_in = x.shape
    e = edge_index.shape[1]
    f_mid = W1.shape[1]
    assert e % (NC * NS * ROWW * G) == 0

    xp = jnp.zeros((NP, D), jnp.float32).at[:n, :f_in].set(x)
    z = jnp.zeros((NP, D), jnp.float32)

    acc1 = _sc_segment_sum(xp, edge_index, z)

    w1p = jnp.zeros((D, f_mid), jnp.float32).at[:f_in].set(W1)
    w2c = jnp.zeros((f_mid, 4), jnp.float32).at[:, :3].set(W2)
    ws = jnp.concatenate([w1p.reshape(-1), b1, w2c.reshape(-1)])
    wsplat = jnp.tile(ws[:, None], (1, 16))
    p = _sc_dense(acc1, wsplat, f_in, f_mid)

    acc2 = _sc_segment_sum(p, edge_index, z)

    bpat = jnp.tile(jnp.concatenate([b2, jnp.zeros((5,), jnp.float32)]), 2)
    out_flat = _sc_finish(acc2, bpat)
    return out_flat.reshape(NP, 3)[:n]
